# baseline (speedup 1.0000x reference)
import dataclasses
import functools

import jax
import jax.numpy as jnp
from jax import lax
from jax.experimental import pallas as pl
from jax.experimental.pallas import tpu as pltpu
from jax.experimental.pallas import tpu_sc as plsc

L = 16
NV = 64
S = NV * L
HV = 32


def _srt(v, desc):
    if desc:
        return plsc.sort_key_val(v, v, descending=True)[0]
    return lax.sort(v, dimension=0, is_stable=False)


def _cmpx(b, i, j, desc):
    a, c = b[i], b[j]
    if desc:
        b[i] = jnp.maximum(a, c)
        b[j] = jnp.minimum(a, c)
    else:
        b[i] = jnp.minimum(a, c)
        b[j] = jnp.maximum(a, c)


def _merge_level(b, lv, desc_fn):
    n = len(b)
    d = lv // 2
    while d >= 1:
        for t in range(0, n, 2 * d):
            de = desc_fn(t // lv)
            for i in range(d):
                _cmpx(b, t + i, t + d + i, de)
        d //= 2
    b[:] = [_srt(b[i], desc_fn(i // lv)) for i in range(n)]


def _bottom32(b, h_even):
    b[:] = [_srt(b[i], desc=(i % 2 == 0)) for i in range(HV)]
    for lv in (2, 4, 8, 16):
        _merge_level(b, lv, lambda r: (r % 2 == 0))
    _merge_level(b, HV, lambda r: h_even)


def kernel(x, W, b):
    B, C, H, Wd = x.shape
    assert H * Wd == S
    xf = x.reshape(B, C, S)

    mesh = plsc.VectorSubcoreMesh(core_axis_name="core", subcore_axis_name="subcore")
    NW = mesh.num_cores * mesh.num_subcores
    CPW = C // NW

    cp = pltpu.CompilerParams()
    if "needs_layout_passes" in pltpu.CompilerParams.__dataclass_fields__:
        cp = dataclasses.replace(cp, needs_layout_passes=False)

    @functools.partial(
        pl.kernel,
        out_type=jax.ShapeDtypeStruct((C, B), jnp.float32),
        mesh=mesh,
        compiler_params=cp,
        scratch_types=[
            pltpu.VMEM((CPW, S), jnp.float32),
            pltpu.VMEM((CPW, S), jnp.float32),
            pltpu.VMEM((CPW, S), jnp.float32),
            pltpu.VMEM((CPW,), jnp.float32),
            pltpu.VMEM((CPW, B), jnp.float32),
            pltpu.SemaphoreType.DMA,
            pltpu.SemaphoreType.DMA,
        ],
    )
    def grp(x_hbm, w_hbm, b_hbm, o_hbm, xb0, xb1, wch, bch, obuf, sem0, sem1):
        lane_iota = lax.iota(jnp.int32, L)
        cid = lax.axis_index("core")
        sid = lax.axis_index("subcore")
        wid = sid * mesh.num_cores + cid
        wc = wid * CPW

        pltpu.sync_copy(w_hbm.at[pl.ds(wc, CPW), :], wch)
        pltpu.sync_copy(b_hbm.at[pl.ds(wc, CPW)], bch)

        def process2(xb, bidx):
            @plsc.parallel_loop(0, CPW, step=2)
            def _(cl0):
              for _k in range(2):
                cl = cl0 + _k
                for h, he in ((0, True), (1, False)):
                    base = h * HV * L
                    blk = [xb[cl, pl.ds(base + i * L, L)] for i in range(HV)]
                    _bottom32(blk, he)
                    for i in range(HV):
                        xb[cl, pl.ds(base + i * L, L)] = blk[i]

                @plsc.parallel_loop(0, HV, step=8)
                def _(i0):
                    for k in range(8):
                        i = i0 + k
                        lo = i * L
                        hi = (i + HV) * L
                        a = xb[cl, pl.ds(lo, L)]
                        c = xb[cl, pl.ds(hi, L)]
                        xb[cl, pl.ds(lo, L)] = jnp.maximum(a, c)
                        xb[cl, pl.ds(hi, L)] = jnp.minimum(a, c)

                acc = jnp.zeros((L,), jnp.float32)
                for h in (0, 1):
                    base = h * HV * L
                    blk = [xb[cl, pl.ds(base + i * L, L)] for i in range(HV)]
                    _merge_level(blk, HV, lambda r: True)
                    for i in range(HV):
                        acc = acc + blk[i] * wch[cl, pl.ds(base + i * L, L)]

                total = jnp.sum(acc)
                bias_vec = plsc.load_gather(bch, [jnp.full((L,), cl, jnp.int32)])
                res = total + bias_vec
                plsc.store_scatter(
                    obuf,
                    [jnp.full((L,), cl, jnp.int32), jnp.full((L,), bidx, jnp.int32)],
                    res,
                    mask=lane_iota == 0,
                )

        pltpu.async_copy(x_hbm.at[0, pl.ds(wc, CPW), :], xb0, sem0)

        @pl.loop(0, B, step=2)
        def _(bb):
            pltpu.async_copy(x_hbm.at[bb + 1, pl.ds(wc, CPW), :], xb1, sem1)
            pltpu.make_async_copy(x_hbm.at[0, pl.ds(wc, CPW), :], xb0, sem0).wait()
            process2(xb0, bb)

            @pl.when(bb + 2 < B)
            def _():
                pltpu.async_copy(x_hbm.at[bb + 2, pl.ds(wc, CPW), :], xb0, sem0)

            pltpu.make_async_copy(x_hbm.at[0, pl.ds(wc, CPW), :], xb1, sem1).wait()
            process2(xb1, bb + 1)

        pltpu.sync_copy(obuf, o_hbm.at[pl.ds(wc, CPW), :])

    out_t = grp(xf, W, b)
    return out_t.T


# speedup vs baseline: 2.9792x; 2.9792x over previous
import dataclasses
import functools

import jax
import jax.numpy as jnp
from jax import lax
from jax.experimental import pallas as pl
from jax.experimental.pallas import tpu as pltpu
from jax.experimental.pallas import tpu_sc as plsc

L = 16
NV = 64
S = NV * L
HV = 32


def _srt(v, desc):
    if desc:
        return plsc.sort_key_val(v, v, descending=True)[0]
    return lax.sort(v, dimension=0, is_stable=False)


def _cmpx(b, i, j, desc):
    a, c = b[i], b[j]
    if desc:
        b[i] = jnp.maximum(a, c)
        b[j] = jnp.minimum(a, c)
    else:
        b[i] = jnp.minimum(a, c)
        b[j] = jnp.maximum(a, c)


def _merge_level(b, lv, desc_fn):
    n = len(b)
    d = lv // 2
    while d >= 1:
        for t in range(0, n, 2 * d):
            de = desc_fn(t // lv)
            for i in range(d):
                _cmpx(b, t + i, t + d + i, de)
        d //= 2
    b[:] = [_srt(b[i], desc_fn(i // lv)) for i in range(n)]


def _bottom32(b, h_even):
    b[:] = [_srt(b[i], desc=(i % 2 == 0)) for i in range(HV)]
    for lv in (2, 4, 8, 16):
        _merge_level(b, lv, lambda r: (r % 2 == 0))
    _merge_level(b, HV, lambda r: h_even)


def kernel(x, W, b):
    B, C, H, Wd = x.shape
    assert H * Wd == S
    xf = x.reshape(B, C, S)

    mesh = plsc.VectorSubcoreMesh(core_axis_name="core", subcore_axis_name="subcore")
    NW = mesh.num_cores * mesh.num_subcores
    CPW = C // NW

    cp = pltpu.CompilerParams()
    if "needs_layout_passes" in pltpu.CompilerParams.__dataclass_fields__:
        cp = dataclasses.replace(cp, needs_layout_passes=False)

    @functools.partial(
        pl.kernel,
        out_type=jax.ShapeDtypeStruct((C, B), jnp.float32),
        mesh=mesh,
        compiler_params=cp,
        scratch_types=[
            pltpu.VMEM((CPW, S), jnp.float32),
            pltpu.VMEM((CPW, S), jnp.float32),
            pltpu.VMEM((CPW, S), jnp.float32),
            pltpu.VMEM((CPW,), jnp.float32),
            pltpu.VMEM((CPW, B), jnp.float32),
            pltpu.SemaphoreType.DMA,
            pltpu.SemaphoreType.DMA,
        ],
    )
    def grp(x_hbm, w_hbm, b_hbm, o_hbm, xb0, xb1, wch, bch, obuf, sem0, sem1):
        lane_iota = lax.iota(jnp.int32, L)
        cid = lax.axis_index("core")
        sid = lax.axis_index("subcore")
        wid = sid * mesh.num_cores + cid
        wc = wid * CPW

        pltpu.sync_copy(w_hbm.at[pl.ds(wc, CPW), :], wch)
        pltpu.sync_copy(b_hbm.at[pl.ds(wc, CPW)], bch)

        def process2(xb, bidx):
            @plsc.parallel_loop(0, CPW)
            def _(cl):
                blk = [xb[cl, pl.ds(i * L, L)] for i in range(HV)]
                _bottom32(blk, True)
                for i in range(HV):
                    xb[cl, pl.ds(i * L, L)] = blk[i]

                base1 = HV * L
                blk = [xb[cl, pl.ds(base1 + i * L, L)] for i in range(HV)]
                _bottom32(blk, False)

                for i in range(HV):
                    a = xb[cl, pl.ds(i * L, L)]
                    xb[cl, pl.ds(i * L, L)] = jnp.maximum(a, blk[i])
                    blk[i] = jnp.minimum(a, blk[i])

                acc = jnp.zeros((L,), jnp.float32)
                _merge_level(blk, HV, lambda r: True)
                for i in range(HV):
                    acc = acc + blk[i] * wch[cl, pl.ds(base1 + i * L, L)]

                blk = [xb[cl, pl.ds(i * L, L)] for i in range(HV)]
                _merge_level(blk, HV, lambda r: True)
                for i in range(HV):
                    acc = acc + blk[i] * wch[cl, pl.ds(i * L, L)]

                total = jnp.sum(acc)
                bias_vec = plsc.load_gather(bch, [jnp.full((L,), cl, jnp.int32)])
                res = total + bias_vec
                plsc.store_scatter(
                    obuf,
                    [jnp.full((L,), cl, jnp.int32), jnp.full((L,), bidx, jnp.int32)],
                    res,
                    mask=lane_iota == 0,
                )

        pltpu.async_copy(x_hbm.at[0, pl.ds(wc, CPW), :], xb0, sem0)

        @pl.loop(0, B, step=2)
        def _(bb):
            pltpu.async_copy(x_hbm.at[bb + 1, pl.ds(wc, CPW), :], xb1, sem1)
            pltpu.make_async_copy(x_hbm.at[0, pl.ds(wc, CPW), :], xb0, sem0).wait()
            process2(xb0, bb)

            @pl.when(bb + 2 < B)
            def _():
                pltpu.async_copy(x_hbm.at[bb + 2, pl.ds(wc, CPW), :], xb0, sem0)

            pltpu.make_async_copy(x_hbm.at[0, pl.ds(wc, CPW), :], xb1, sem1).wait()
            process2(xb1, bb + 1)

        pltpu.sync_copy(obuf, o_hbm.at[pl.ds(wc, CPW), :])

    out_t = grp(xf, W, b)
    return out_t.T
